# Initial kernel scaffold; baseline (speedup 1.0000x reference)
import jax
import jax.numpy as jnp
from jax.experimental import pallas as pl


def kernel(feat0, feat1, adj0, adj1, params):
    raise NotImplementedError("write your pallas kernel here")


# speedup vs baseline: 1.0439x; 1.0439x over previous
import functools

import jax
import jax.numpy as jnp
from jax.experimental import pallas as pl
from jax.experimental.pallas import tpu as pltpu

N = 4096
F32 = jnp.float32


def _dot(a, b):
    return jnp.dot(a, b, preferred_element_type=F32)


def _dot_t(a, b):
    return jax.lax.dot_general(a, b, (((1,), (1,)), ((), ())),
                               preferred_element_type=F32)


def _mm_body(x_ref, w_ref, o_ref):
    o_ref[...] = _dot(x_ref[...], w_ref[...])


def _feat_mm(x, w, bm):
    n, f = x.shape
    dout = w.shape[1]
    return pl.pallas_call(
        _mm_body,
        grid=(n // bm,),
        in_specs=[
            pl.BlockSpec((bm, f), lambda i: (i, 0)),
            pl.BlockSpec((f, dout), lambda i: (0, 0)),
        ],
        out_specs=pl.BlockSpec((bm, dout), lambda i: (i, 0)),
        out_shape=jax.ShapeDtypeStruct((n, dout), F32),
    )(x, w)


def _gcn1_body(adj_ref, t1_ref, b1_ref, w2_ref, o_ref):
    u = _dot(adj_ref[...], t1_ref[...]) + b1_ref[...]
    u = jnp.maximum(u, 0.0)
    o_ref[...] = _dot(u, w2_ref[...])


def _gcn1(adj, t1, b1, w2, bm):
    h1 = t1.shape[1]
    h2 = w2.shape[1]
    return pl.pallas_call(
        _gcn1_body,
        grid=(N // bm,),
        in_specs=[
            pl.BlockSpec((bm, N), lambda i: (i, 0)),
            pl.BlockSpec((N, h1), lambda i: (0, 0)),
            pl.BlockSpec((1, h1), lambda i: (0, 0)),
            pl.BlockSpec((h1, h2), lambda i: (0, 0)),
        ],
        out_specs=pl.BlockSpec((bm, h2), lambda i: (i, 0)),
        out_shape=jax.ShapeDtypeStruct((N, h2), F32),
    )(adj, t1, b1, w2)


def _gcn2_body(adj_ref, t2_ref, b2_ref, wfg_ref, hp_ref, y_ref):
    hp = _dot(adj_ref[...], t2_ref[...]) + b2_ref[...]
    hp_ref[...] = hp
    y_ref[...] = _dot(hp, wfg_ref[...])


def _gcn2(adj, t2, b2, wfg, bm):
    h2 = t2.shape[1]
    return pl.pallas_call(
        _gcn2_body,
        grid=(N // bm,),
        in_specs=[
            pl.BlockSpec((bm, N), lambda i: (i, 0)),
            pl.BlockSpec((N, h2), lambda i: (0, 0)),
            pl.BlockSpec((1, h2), lambda i: (0, 0)),
            pl.BlockSpec((h2, h2), lambda i: (0, 0)),
        ],
        out_specs=[
            pl.BlockSpec((bm, h2), lambda i: (i, 0)),
            pl.BlockSpec((bm, h2), lambda i: (i, 0)),
        ],
        out_shape=[
            jax.ShapeDtypeStruct((N, h2), F32),
            jax.ShapeDtypeStruct((N, h2), F32),
        ],
    )(adj, t2, b2, wfg)


def _fg_body(hpi_ref, hpj_ref, yj_ref, yo_ref, bfg_ref, o_ref, acc_ref, *, nj):
    j = pl.program_id(1)

    @pl.when(j == 0)
    def _():
        acc_ref[...] = jnp.zeros_like(acc_ref)

    s = jax.nn.sigmoid(_dot_t(hpi_ref[...], hpj_ref[...]))
    acc_ref[...] += _dot(s, yj_ref[...])

    @pl.when(j == nj - 1)
    def _():
        o_ref[...] = acc_ref[...] + yo_ref[...] + bfg_ref[...]


def _fg_layer(hp, y, y_other, bfg, bm, bn):
    h2 = hp.shape[1]
    nj = N // bn
    return pl.pallas_call(
        functools.partial(_fg_body, nj=nj),
        grid=(N // bm, nj),
        in_specs=[
            pl.BlockSpec((bm, h2), lambda i, j: (i, 0)),
            pl.BlockSpec((bn, h2), lambda i, j: (j, 0)),
            pl.BlockSpec((bn, h2), lambda i, j: (j, 0)),
            pl.BlockSpec((bm, h2), lambda i, j: (i, 0)),
            pl.BlockSpec((1, h2), lambda i, j: (0, 0)),
        ],
        out_specs=pl.BlockSpec((bm, h2), lambda i, j: (i, 0)),
        out_shape=jax.ShapeDtypeStruct((N, h2), F32),
        scratch_shapes=[pltpu.VMEM((bm, h2), F32)],
    )(hp, hp, y, y_other, bfg)


def _att_body(h0_ref, h1_ref, w1a_ref, b1a_ref, w2a_ref, c_ref,
              d1_0_ref, db1_0_ref, d2_0_ref, db2_0_ref,
              d1_1_ref, db1_1_ref, d2_1_ref, db2_1_ref,
              z_ref, q_ref, *xz_refs, with_xz):
    h0 = h0_ref[...]
    h1 = h1_ref[...]
    w1a = w1a_ref[...]
    b1a = b1a_ref[...]
    w2a = w2a_ref[...]
    s0 = _dot(jnp.maximum(_dot(h0, w1a) + b1a, 0.0), w2a)
    s1 = _dot(jnp.maximum(_dot(h1, w1a) + b1a, 0.0), w2a)
    m = jnp.maximum(s0, s1)
    e0 = jnp.exp(s0 - m)
    e1 = jnp.exp(s1 - m)
    z = (e0 * h0 + e1 * h1) / (e0 + e1)
    z_ref[...] = z

    c = c_ref[...]
    zz = jnp.sum(z * z, axis=1, keepdims=True)
    cc = jnp.sum(c * c, axis=1)[None, :]
    cross = _dot_t(z, c)
    q = 1.0 / (1.0 + zz + cc - 2.0 * cross)
    q_ref[...] = q / jnp.sum(q, axis=1, keepdims=True)

    if with_xz:
        x0_ref, x1_ref = xz_refs
        x = jnp.maximum(_dot(z, d1_0_ref[...]) + db1_0_ref[...], 0.0)
        x0_ref[...] = _dot(x, d2_0_ref[...]) + db2_0_ref[...]
        x = jnp.maximum(_dot(z, d1_1_ref[...]) + db1_1_ref[...], 0.0)
        x1_ref[...] = _dot(x, d2_1_ref[...]) + db2_1_ref[...]


def _attention_fused(h0, h1, att, c, dec, bm, with_xz):
    h2 = h0.shape[1]
    kc = c.shape[0]
    (w1a, b1a, w2a) = att
    (d1_0, db1_0, d2_0, db2_0, d1_1, db1_1, d2_1, db2_1) = dec
    f0 = d2_0.shape[1]
    f1 = d2_1.shape[1]
    hmid = d1_0.shape[1]
    out_shape = [
        jax.ShapeDtypeStruct((N, h2), F32),
        jax.ShapeDtypeStruct((N, kc), F32),
        jax.ShapeDtypeStruct((N, f0), F32),
        jax.ShapeDtypeStruct((N, f1), F32),
    ]
    out_specs = [
        pl.BlockSpec((bm, h2), lambda i: (i, 0)),
        pl.BlockSpec((bm, kc), lambda i: (i, 0)),
        pl.BlockSpec((bm, f0), lambda i: (i, 0)),
        pl.BlockSpec((bm, f1), lambda i: (i, 0)),
    ]
    if not with_xz:
        out_shape = out_shape[:2]
        out_specs = out_specs[:2]
    outs = pl.pallas_call(
        functools.partial(_att_body, with_xz=with_xz),
        grid=(N // bm,),
        in_specs=[
            pl.BlockSpec((bm, h2), lambda i: (i, 0)),
            pl.BlockSpec((bm, h2), lambda i: (i, 0)),
            pl.BlockSpec((h2, h2), lambda i: (0, 0)),
            pl.BlockSpec((1, h2), lambda i: (0, 0)),
            pl.BlockSpec((h2, 1), lambda i: (0, 0)),
            pl.BlockSpec((kc, h2), lambda i: (0, 0)),
            pl.BlockSpec((h2, hmid), lambda i: (0, 0)),
            pl.BlockSpec((1, hmid), lambda i: (0, 0)),
            pl.BlockSpec((hmid, f0), lambda i: (0, 0)),
            pl.BlockSpec((1, f0), lambda i: (0, 0)),
            pl.BlockSpec((h2, hmid), lambda i: (0, 0)),
            pl.BlockSpec((1, hmid), lambda i: (0, 0)),
            pl.BlockSpec((hmid, f1), lambda i: (0, 0)),
            pl.BlockSpec((1, f1), lambda i: (0, 0)),
        ],
        out_specs=out_specs,
        out_shape=out_shape,
    )(h0, h1, w1a, b1a, w2a, c,
      d1_0, db1_0, d2_0, db2_0, d1_1, db1_1, d2_1, db2_1)
    if with_xz:
        return outs
    return outs[0], outs[1]


def _adjz_body(zi_ref, zj_ref, o_ref):
    o_ref[...] = jax.nn.sigmoid(_dot_t(zi_ref[...], zj_ref[...]))


def _adjz(z, bm, bn):
    h2 = z.shape[1]
    return pl.pallas_call(
        _adjz_body,
        grid=(N // bm, N // bn),
        in_specs=[
            pl.BlockSpec((bm, h2), lambda i, j: (i, 0)),
            pl.BlockSpec((bn, h2), lambda i, j: (j, 0)),
        ],
        out_specs=pl.BlockSpec((bm, bn), lambda i, j: (i, j)),
        out_shape=jax.ShapeDtypeStruct((N, N), F32),
    )(z, z)


@jax.jit
def kernel(feat0, feat1, adj0, adj1, params):
    feats = (feat0, feat1)
    adjs = (adj0, adj1)
    wfg, bfg = params["fg"]
    bfg = bfg.reshape(1, -1)
    w1a, b1a, w2a = params["att"]
    att = (w1a, b1a.reshape(1, -1), w2a)
    c = params["cluster"]
    dec_flat = []
    for v in range(2):
        (d1, db1), (d2, db2) = params["dec"][v]
        dec_flat += [d1, db1.reshape(1, -1), d2, db2.reshape(1, -1)]
    dec_flat = tuple(dec_flat)

    hps = []
    ys = []
    for v in range(2):
        (w1, b1), (w2, b2) = params["enc"][v]
        t1 = _feat_mm(feats[v], w1, bm=512)
        t2 = _gcn1(adjs[v], t1, b1.reshape(1, -1), w2, bm=512)
        hp, y = _gcn2(adjs[v], t2, b2.reshape(1, -1), wfg, bm=512)
        hps.append(hp)
        ys.append(y)

    z, qz, xz0, xz1 = _attention_fused(hps[0], hps[1], att, c, dec_flat,
                                       bm=512, with_xz=True)

    h_all0 = _fg_layer(hps[0], ys[0], ys[1], bfg, bm=512, bn=512)
    h_all1 = _fg_layer(hps[1], ys[1], ys[0], bfg, bm=512, bn=512)
    h, qh = _attention_fused(h_all0, h_all1, att, c, dec_flat,
                             bm=512, with_xz=False)

    adjz = _adjz(z, bm=512, bn=512)
    return (h, z, adjz, xz0, xz1, qz, qh)
